# Initial kernel scaffold; baseline (speedup 1.0000x reference)
import jax
import jax.numpy as jnp
from jax.experimental import pallas as pl


def kernel(y_true, bbox_true, conf_pred, logit_pred, bbox_pred, anchors):
    raise NotImplementedError("write your pallas kernel here")


# speedup vs baseline: 1.7638x; 1.7638x over previous
import functools

import jax
import jax.numpy as jnp
import numpy as np
from jax.experimental import pallas as pl

NUM_CLASSES = 80
NUM_ANCHORS = 20000
BATCH = 8
MAX_TRUE = 100
POS_THRESH = 0.5
NEG_THRESH = 0.4
ALPHA = 0.25
GAMMA = 2.0
EPS = 1e-7

AB = 2000
NBLK = NUM_ANCHORS // AB

_ATAN_C = (1.00000000e+00, -3.33333331e-01, 1.99999846e-01, -1.42853316e-01,
           1.11062643e-01, -9.05436302e-02, 7.51323369e-02, -6.06349744e-02,
           4.42830421e-02, -2.66563540e-02, 1.18503795e-02, -3.35367563e-03,
           4.45197908e-04)


def _atan_nonneg(x):
    big = x > 1.0
    t = jnp.where(big, 1.0 / jnp.maximum(x, 1.0), x)
    t2 = t * t
    acc = jnp.full_like(t, _ATAN_C[-1])
    for c in _ATAN_C[-2::-1]:
        acc = acc * t2 + c
    a = t * acc
    return jnp.where(big, (np.pi / 2.0) - a, a)


def _loss_block(anc_ref, bt_ref, ytT_ref, conf_ref, logit_ref, bp_ref, out_ref):
    j = pl.program_id(1)
    anc = anc_ref[...]
    bt = bt_ref[0]
    ytT = ytT_ref[0]
    conf = conf_ref[0]
    q_raw = logit_ref[0]
    bp = bp_ref[0]

    ax1 = anc[:, 0:1]; ay1 = anc[:, 1:2]; ax2 = anc[:, 2:3]; ay2 = anc[:, 3:4]
    bx1 = bt[0:1, :]; by1 = bt[1:2, :]; bx2 = bt[2:3, :]; by2 = bt[3:4, :]

    ix1 = jnp.maximum(ax1, bx1)
    iy1 = jnp.maximum(ay1, by1)
    ix2 = jnp.minimum(ax2, bx2)
    iy2 = jnp.minimum(ay2, by2)
    inter = jnp.clip(ix2 - ix1, 0.0) * jnp.clip(iy2 - iy1, 0.0)
    area_a = jnp.clip(ax2 - ax1, 0.0) * jnp.clip(ay2 - ay1, 0.0)
    area_b = jnp.clip(bx2 - bx1, 0.0) * jnp.clip(by2 - by1, 0.0)
    iou = inter / (area_a + area_b - inter + EPS)

    valid = jnp.any(bt > 0, axis=0, keepdims=True)
    iou = jnp.where(valid, iou, -1.0)
    max_iou = jnp.max(iou, axis=1, keepdims=True)
    tidx = jax.lax.broadcasted_iota(jnp.int32, (AB, MAX_TRUE), 1)
    argf = jnp.min(jnp.where(iou == max_iou, tidx, jnp.int32(1 << 30)),
                   axis=1, keepdims=True)
    sel = (tidx == argf).astype(jnp.float32)

    pos = max_iou >= POS_THRESH
    neg = max_iou < NEG_THRESH
    pw = pos.astype(jnp.float32)
    tw = (pos | neg).astype(jnp.float32)

    p = jnp.clip(conf, EPS, 1.0 - EPS)
    bce = -(pw * jnp.log(p) + (1.0 - pw) * jnp.log(1.0 - p))
    s_sum = jnp.sum(bce * tw)

    q = jnp.clip(q_raw, EPS, 1.0 - EPS)
    neg_term = (1.0 - ALPHA) * q * q * (-jnp.log(1.0 - q))
    s_neg = jnp.sum(neg_term, axis=1, keepdims=True)
    k_iota = jax.lax.broadcasted_iota(
        jnp.int32, (NUM_CLASSES, MAX_TRUE), 0).astype(jnp.float32)
    cls_row = jnp.sum(ytT * k_iota, axis=0, keepdims=True)
    c_a = jnp.sum(sel * cls_row, axis=1, keepdims=True)
    lane_k = jax.lax.broadcasted_iota(
        jnp.int32, (AB, NUM_CLASSES), 1).astype(jnp.float32)
    onehot = (lane_k == c_a).astype(jnp.float32)
    qc = jnp.sum(q * onehot, axis=1, keepdims=True)
    pos_term_c = ALPHA * (1.0 - qc) * (1.0 - qc) * (-jnp.log(qc))
    neg_term_c = (1.0 - ALPHA) * qc * qc * (-jnp.log(1.0 - qc))
    c_sum = jnp.sum(pw * (s_neg - neg_term_c + pos_term_c))

    x1t = jnp.sum(sel * bx1, axis=1, keepdims=True)
    y1t = jnp.sum(sel * by1, axis=1, keepdims=True)
    x2t = jnp.sum(sel * bx2, axis=1, keepdims=True)
    y2t = jnp.sum(sel * by2, axis=1, keepdims=True)
    x1t = x1t * pw; y1t = y1t * pw; x2t = x2t * pw; y2t = y2t * pw
    x1p = bp[:, 0:1]; y1p = bp[:, 1:2]; x2p = bp[:, 2:3]; y2p = bp[:, 3:4]
    wt = jnp.clip(x2t - x1t, 0.0); ht = jnp.clip(y2t - y1t, 0.0)
    wp = jnp.clip(x2p - x1p, 0.0); hp = jnp.clip(y2p - y1p, 0.0)
    inter2 = jnp.clip(jnp.minimum(x2t, x2p) - jnp.maximum(x1t, x1p), 0.0) * \
             jnp.clip(jnp.minimum(y2t, y2p) - jnp.maximum(y1t, y1p), 0.0)
    union = wt * ht + wp * hp - inter2
    iou2 = inter2 / (union + EPS)
    cw = jnp.maximum(x2t, x2p) - jnp.minimum(x1t, x1p)
    ch = jnp.maximum(y2t, y2p) - jnp.minimum(y1t, y1p)
    c2 = cw * cw + ch * ch + EPS
    rho2 = ((x1t + x2t - x1p - x2p) ** 2 + (y1t + y2t - y1p - y2p) ** 2) / 4.0
    v = (4.0 / (np.pi ** 2)) * (_atan_nonneg(wt / (ht + EPS)) -
                                _atan_nonneg(wp / (hp + EPS))) ** 2
    alpha_t = v / (1.0 - iou2 + v + EPS)
    cl = 1.0 - (iou2 - rho2 / c2 - alpha_t * v)
    b_sum = jnp.sum(cl * pw)

    cnt = jnp.sum(pw)

    rows = jnp.concatenate([
        jnp.full((1, 128), s_sum, jnp.float32),
        jnp.full((1, 128), c_sum, jnp.float32),
        jnp.full((1, 128), b_sum, jnp.float32),
        jnp.full((1, 128), cnt, jnp.float32),
    ], axis=0)

    @pl.when(j == 0)
    def _init():
        out_ref[0] = rows

    @pl.when(j != 0)
    def _acc():
        out_ref[0] = out_ref[0] + rows


@jax.jit
def kernel(y_true, bbox_true, conf_pred, logit_pred, bbox_pred, anchors):
    btT = jnp.transpose(bbox_true, (0, 2, 1))
    ytT = jnp.transpose(y_true, (0, 2, 1))

    out = pl.pallas_call(
        _loss_block,
        grid=(BATCH, NBLK),
        in_specs=[
            pl.BlockSpec((AB, 4), lambda b, j: (j, 0)),
            pl.BlockSpec((1, 4, MAX_TRUE), lambda b, j: (b, 0, 0)),
            pl.BlockSpec((1, NUM_CLASSES, MAX_TRUE), lambda b, j: (b, 0, 0)),
            pl.BlockSpec((1, AB, 1), lambda b, j: (b, j, 0)),
            pl.BlockSpec((1, AB, NUM_CLASSES), lambda b, j: (b, j, 0)),
            pl.BlockSpec((1, AB, 4), lambda b, j: (b, j, 0)),
        ],
        out_specs=pl.BlockSpec((1, 4, 128), lambda b, j: (b, 0, 0)),
        out_shape=jax.ShapeDtypeStruct((BATCH, 4, 128), jnp.float32),
    )(anchors, btT, ytT, conf_pred, logit_pred, bbox_pred)

    sums = out[:, :, 0]
    avg = jnp.sum(jnp.maximum(sums[:, 3], 1.0))
    losses = jnp.stack([jnp.sum(sums[:, 0]), jnp.sum(sums[:, 1]),
                        jnp.sum(sums[:, 2])]) / avg
    return jnp.where(jnp.isnan(losses) | jnp.isinf(losses), 0.0, losses)


# speedup vs baseline: 9.0071x; 5.1068x over previous
import jax
import jax.numpy as jnp
import numpy as np
from jax.experimental import pallas as pl

NUM_CLASSES = 80
NUM_ANCHORS = 20000
BATCH = 8
MAX_TRUE = 100
POS_THRESH = 0.5
NEG_THRESH = 0.4
ALPHA = 0.25
GAMMA = 2.0
EPS = 1e-7

AB = 2000
NBLK = NUM_ANCHORS // AB

_ATAN_C = (1.00000000e+00, -3.33333331e-01, 1.99999846e-01, -1.42853316e-01,
           1.11062643e-01, -9.05436302e-02, 7.51323369e-02, -6.06349744e-02,
           4.42830421e-02, -2.66563540e-02, 1.18503795e-02, -3.35367563e-03,
           4.45197908e-04)


def _atan_nonneg(x):
    big = x > 1.0
    t = jnp.where(big, 1.0 / jnp.maximum(x, 1.0), x)
    t2 = t * t
    acc = jnp.full_like(t, _ATAN_C[-1])
    for c in _ATAN_C[-2::-1]:
        acc = acc * t2 + c
    a = t * acc
    return jnp.where(big, (np.pi / 2.0) - a, a)


def _loss_block(ancT_ref, bt_ref, btT_ref, yt_ref, conf_ref, logit_ref,
                bpT_ref, out_ref):
    j = pl.program_id(1)
    anc = ancT_ref[...]
    bt = bt_ref[0]
    btT = btT_ref[0]
    yt = yt_ref[0]
    conf = conf_ref[0, 0]
    q_raw = logit_ref[0]
    bp = bpT_ref[0]

    ax1 = anc[0, 0]; ay1 = anc[1, 0]; ax2 = anc[2, 0]; ay2 = anc[3, 0]
    bx1 = bt[:, 0:1]; by1 = bt[:, 1:2]; bx2 = bt[:, 2:3]; by2 = bt[:, 3:4]

    ix1 = jnp.maximum(ax1, bx1)
    iy1 = jnp.maximum(ay1, by1)
    ix2 = jnp.minimum(ax2, bx2)
    iy2 = jnp.minimum(ay2, by2)
    inter = jnp.clip(ix2 - ix1, 0.0) * jnp.clip(iy2 - iy1, 0.0)
    area_a = jnp.clip(ax2 - ax1, 0.0) * jnp.clip(ay2 - ay1, 0.0)
    area_b = jnp.clip(bx2 - bx1, 0.0) * jnp.clip(by2 - by1, 0.0)
    iou = inter / (area_a + area_b - inter + EPS)

    valid = jnp.any(bt > 0, axis=1, keepdims=True)
    iou = jnp.where(valid, iou, -1.0)
    max_iou = jnp.max(iou, axis=0, keepdims=True)

    pos = max_iou >= POS_THRESH
    neg = max_iou < NEG_THRESH
    pw = pos.astype(jnp.float32)
    tw = (pos | neg).astype(jnp.float32)

    possel = (iou == max_iou).astype(jnp.float32) * pw

    p = jnp.clip(conf, EPS, 1.0 - EPS)
    bce = -(pw * jnp.log(p) + (1.0 - pw) * jnp.log(1.0 - p))
    s_sum = jnp.sum(bce * tw)

    q = jnp.clip(q_raw, EPS, 1.0 - EPS)
    r = 1.0 - q
    neg_term = -(1.0 - ALPHA) * q * q * jnp.log(r)
    pos_term = -ALPHA * r * r * jnp.log(q)
    h = pos_term - neg_term
    t1 = jnp.sum(jnp.dot(pw, neg_term, preferred_element_type=jnp.float32))
    g = jnp.dot(possel, h, preferred_element_type=jnp.float32)
    t2 = jnp.sum(g * yt)
    c_sum = t1 + t2

    basn = jnp.dot(btT, possel, preferred_element_type=jnp.float32)
    x1t = basn[0:1, :]; y1t = basn[1:2, :]; x2t = basn[2:3, :]; y2t = basn[3:4, :]
    x1p = bp[0, 0]; y1p = bp[1, 0]; x2p = bp[2, 0]; y2p = bp[3, 0]
    wt = jnp.clip(x2t - x1t, 0.0); ht = jnp.clip(y2t - y1t, 0.0)
    wp = jnp.clip(x2p - x1p, 0.0); hp = jnp.clip(y2p - y1p, 0.0)
    inter2 = jnp.clip(jnp.minimum(x2t, x2p) - jnp.maximum(x1t, x1p), 0.0) * \
             jnp.clip(jnp.minimum(y2t, y2p) - jnp.maximum(y1t, y1p), 0.0)
    union = wt * ht + wp * hp - inter2
    iou2 = inter2 / (union + EPS)
    cw = jnp.maximum(x2t, x2p) - jnp.minimum(x1t, x1p)
    ch = jnp.maximum(y2t, y2p) - jnp.minimum(y1t, y1p)
    c2 = cw * cw + ch * ch + EPS
    rho2 = ((x1t + x2t - x1p - x2p) ** 2 + (y1t + y2t - y1p - y2p) ** 2) / 4.0
    v = (4.0 / (np.pi ** 2)) * (_atan_nonneg(wt / (ht + EPS)) -
                                _atan_nonneg(wp / (hp + EPS))) ** 2
    alpha_t = v / (1.0 - iou2 + v + EPS)
    cl = 1.0 - (iou2 - rho2 / c2 - alpha_t * v)
    b_sum = jnp.sum(cl * pw)

    cnt = jnp.sum(pw)

    rows = jnp.concatenate([
        jnp.full((1, 128), s_sum, jnp.float32),
        jnp.full((1, 128), c_sum, jnp.float32),
        jnp.full((1, 128), b_sum, jnp.float32),
        jnp.full((1, 128), cnt, jnp.float32),
    ], axis=0)

    @pl.when(j == 0)
    def _init():
        out_ref[0] = rows

    @pl.when(j != 0)
    def _acc():
        out_ref[0] = out_ref[0] + rows


@jax.jit
def kernel(y_true, bbox_true, conf_pred, logit_pred, bbox_pred, anchors):
    ancT = jnp.transpose(anchors, (1, 0)).reshape(4, NBLK, 1, AB)
    btT = jnp.transpose(bbox_true, (0, 2, 1))
    conf3 = conf_pred.reshape(BATCH, NBLK, 1, AB)
    bpT = jnp.transpose(bbox_pred, (0, 2, 1)).reshape(BATCH, 4, NBLK, 1, AB)

    out = pl.pallas_call(
        _loss_block,
        grid=(BATCH, NBLK),
        in_specs=[
            pl.BlockSpec((4, 1, 1, AB), lambda b, j: (0, j, 0, 0)),
            pl.BlockSpec((1, MAX_TRUE, 4), lambda b, j: (b, 0, 0)),
            pl.BlockSpec((1, 4, MAX_TRUE), lambda b, j: (b, 0, 0)),
            pl.BlockSpec((1, MAX_TRUE, NUM_CLASSES), lambda b, j: (b, 0, 0)),
            pl.BlockSpec((1, 1, 1, AB), lambda b, j: (b, j, 0, 0)),
            pl.BlockSpec((1, AB, NUM_CLASSES), lambda b, j: (b, j, 0)),
            pl.BlockSpec((1, 4, 1, 1, AB), lambda b, j: (b, 0, j, 0, 0)),
        ],
        out_specs=pl.BlockSpec((1, 4, 128), lambda b, j: (b, 0, 0)),
        out_shape=jax.ShapeDtypeStruct((BATCH, 4, 128), jnp.float32),
    )(ancT, bbox_true, btT, y_true, conf3, logit_pred, bpT)

    sums = out[:, :, 0]
    avg = jnp.sum(jnp.maximum(sums[:, 3], 1.0))
    losses = jnp.stack([jnp.sum(sums[:, 0]), jnp.sum(sums[:, 1]),
                        jnp.sum(sums[:, 2])]) / avg
    return jnp.where(jnp.isnan(losses) | jnp.isinf(losses), 0.0, losses)


# speedup vs baseline: 9.9027x; 1.0994x over previous
import jax
import jax.numpy as jnp
import numpy as np
from jax.experimental import pallas as pl

NUM_CLASSES = 80
NUM_ANCHORS = 20000
BATCH = 8
MAX_TRUE = 100
POS_THRESH = 0.5
NEG_THRESH = 0.4
ALPHA = 0.25
GAMMA = 2.0
EPS = 1e-7

AB = 4000
NBLK = NUM_ANCHORS // AB

_ATAN_C = (1.00000000e+00, -3.33333331e-01, 1.99999846e-01, -1.42853316e-01,
           1.11062643e-01, -9.05436302e-02, 7.51323369e-02, -6.06349744e-02,
           4.42830421e-02, -2.66563540e-02, 1.18503795e-02, -3.35367563e-03,
           4.45197908e-04)


def _atan_nonneg(x):
    big = x > 1.0
    t = jnp.where(big, 1.0 / jnp.maximum(x, 1.0), x)
    t2 = t * t
    acc = jnp.full_like(t, _ATAN_C[-1])
    for c in _ATAN_C[-2::-1]:
        acc = acc * t2 + c
    a = t * acc
    return jnp.where(big, (np.pi / 2.0) - a, a)


def _loss_block(ancT_ref, bt_ref, btT_ref, yt_ref, conf_ref, logit_ref,
                bpT_ref, out_ref):
    j = pl.program_id(1)
    anc = ancT_ref[...]
    bt = bt_ref[0]
    btT = btT_ref[0]
    yt = yt_ref[0]
    conf = conf_ref[0, 0]
    q_raw = logit_ref[0]
    bp = bpT_ref[0]

    ax1 = anc[0, 0]; ay1 = anc[1, 0]; ax2 = anc[2, 0]; ay2 = anc[3, 0]
    bx1 = bt[:, 0:1]; by1 = bt[:, 1:2]; bx2 = bt[:, 2:3]; by2 = bt[:, 3:4]

    ix1 = jnp.maximum(ax1, bx1)
    iy1 = jnp.maximum(ay1, by1)
    ix2 = jnp.minimum(ax2, bx2)
    iy2 = jnp.minimum(ay2, by2)
    inter = jnp.clip(ix2 - ix1, 0.0) * jnp.clip(iy2 - iy1, 0.0)
    area_a = jnp.clip(ax2 - ax1, 0.0) * jnp.clip(ay2 - ay1, 0.0)
    area_b = jnp.clip(bx2 - bx1, 0.0) * jnp.clip(by2 - by1, 0.0)
    iou = inter / (area_a + area_b - inter + EPS)

    valid = jnp.any(bt > 0, axis=1, keepdims=True)
    iou = jnp.where(valid, iou, -1.0)
    max_iou = jnp.max(iou, axis=0, keepdims=True)

    pos = max_iou >= POS_THRESH
    neg = max_iou < NEG_THRESH
    pw = pos.astype(jnp.float32)
    tw = (pos | neg).astype(jnp.float32)

    possel = (iou == max_iou).astype(jnp.float32) * pw

    p = jnp.clip(conf, EPS, 1.0 - EPS)
    bce = -(pw * jnp.log(p) + (1.0 - pw) * jnp.log(1.0 - p))
    s_sum = jnp.sum(bce * tw)

    q = jnp.clip(q_raw, EPS, 1.0 - EPS)
    r = 1.0 - q
    neg_term = -(1.0 - ALPHA) * q * q * jnp.log(r)
    pos_term = -ALPHA * r * r * jnp.log(q)
    h = pos_term - neg_term
    t1 = jnp.sum(jnp.dot(pw, neg_term, preferred_element_type=jnp.float32))
    g = jnp.dot(possel, h, preferred_element_type=jnp.float32)
    t2 = jnp.sum(g * yt)
    c_sum = t1 + t2

    basn = jnp.dot(btT, possel, preferred_element_type=jnp.float32)
    x1t = basn[0:1, :]; y1t = basn[1:2, :]; x2t = basn[2:3, :]; y2t = basn[3:4, :]
    x1p = bp[0, 0]; y1p = bp[1, 0]; x2p = bp[2, 0]; y2p = bp[3, 0]
    wt = jnp.clip(x2t - x1t, 0.0); ht = jnp.clip(y2t - y1t, 0.0)
    wp = jnp.clip(x2p - x1p, 0.0); hp = jnp.clip(y2p - y1p, 0.0)
    inter2 = jnp.clip(jnp.minimum(x2t, x2p) - jnp.maximum(x1t, x1p), 0.0) * \
             jnp.clip(jnp.minimum(y2t, y2p) - jnp.maximum(y1t, y1p), 0.0)
    union = wt * ht + wp * hp - inter2
    iou2 = inter2 / (union + EPS)
    cw = jnp.maximum(x2t, x2p) - jnp.minimum(x1t, x1p)
    ch = jnp.maximum(y2t, y2p) - jnp.minimum(y1t, y1p)
    c2 = cw * cw + ch * ch + EPS
    rho2 = ((x1t + x2t - x1p - x2p) ** 2 + (y1t + y2t - y1p - y2p) ** 2) / 4.0
    v = (4.0 / (np.pi ** 2)) * (_atan_nonneg(wt / (ht + EPS)) -
                                _atan_nonneg(wp / (hp + EPS))) ** 2
    alpha_t = v / (1.0 - iou2 + v + EPS)
    cl = 1.0 - (iou2 - rho2 / c2 - alpha_t * v)
    b_sum = jnp.sum(cl * pw)

    cnt = jnp.sum(pw)

    rows = jnp.concatenate([
        jnp.full((1, 128), s_sum, jnp.float32),
        jnp.full((1, 128), c_sum, jnp.float32),
        jnp.full((1, 128), b_sum, jnp.float32),
        jnp.full((1, 128), cnt, jnp.float32),
    ], axis=0)

    @pl.when(j == 0)
    def _init():
        out_ref[0] = rows

    @pl.when(j != 0)
    def _acc():
        out_ref[0] = out_ref[0] + rows


@jax.jit
def kernel(y_true, bbox_true, conf_pred, logit_pred, bbox_pred, anchors):
    ancT = jnp.transpose(anchors, (1, 0)).reshape(4, NBLK, 1, AB)
    btT = jnp.transpose(bbox_true, (0, 2, 1))
    conf3 = conf_pred.reshape(BATCH, NBLK, 1, AB)
    bpT = jnp.transpose(bbox_pred, (0, 2, 1)).reshape(BATCH, 4, NBLK, 1, AB)

    out = pl.pallas_call(
        _loss_block,
        grid=(BATCH, NBLK),
        in_specs=[
            pl.BlockSpec((4, 1, 1, AB), lambda b, j: (0, j, 0, 0)),
            pl.BlockSpec((1, MAX_TRUE, 4), lambda b, j: (b, 0, 0)),
            pl.BlockSpec((1, 4, MAX_TRUE), lambda b, j: (b, 0, 0)),
            pl.BlockSpec((1, MAX_TRUE, NUM_CLASSES), lambda b, j: (b, 0, 0)),
            pl.BlockSpec((1, 1, 1, AB), lambda b, j: (b, j, 0, 0)),
            pl.BlockSpec((1, AB, NUM_CLASSES), lambda b, j: (b, j, 0)),
            pl.BlockSpec((1, 4, 1, 1, AB), lambda b, j: (b, 0, j, 0, 0)),
        ],
        out_specs=pl.BlockSpec((1, 4, 128), lambda b, j: (b, 0, 0)),
        out_shape=jax.ShapeDtypeStruct((BATCH, 4, 128), jnp.float32),
    )(ancT, bbox_true, btT, y_true, conf3, logit_pred, bpT)

    sums = out[:, :, 0]
    avg = jnp.sum(jnp.maximum(sums[:, 3], 1.0))
    losses = jnp.stack([jnp.sum(sums[:, 0]), jnp.sum(sums[:, 1]),
                        jnp.sum(sums[:, 2])]) / avg
    return jnp.where(jnp.isnan(losses) | jnp.isinf(losses), 0.0, losses)
